# baseline (speedup 1.0000x reference)
import jax
import jax.numpy as jnp
from jax import lax
from jax.experimental import pallas as pl
from jax.experimental.pallas import tpu as pltpu
from jax.experimental.pallas import tpu_sc as plsc

N_NODES = 10000
FEATS = 128
N_EDGES = 320000

NC = 2
NS = 16
NW = NC * NS
CHUNK = 128
EROWS = 2560
TROWS = EROWS // NW
EPAD = EROWS * CHUNK
NPAD = 10112
ROWS_PT = NPAD // NS
TRASH = 10048
NBUF = 4
DNB = 8

_mesh = plsc.VectorSubcoreMesh(core_axis_name="c", subcore_axis_name="s",
                               num_cores=NC, num_subcores=NS)


def _deg_body(dst_hbm, ones_hbm, zeros_hbm, out_hbm, acc, idxd, ones_v, sem):
    cid = lax.axis_index("c")
    sid = lax.axis_index("s")
    wid = cid * NS + sid
    rbase = sid * ROWS_PT

    pltpu.sync_copy(dst_hbm.at[pl.ds(wid * TROWS, TROWS)], idxd)
    pltpu.sync_copy(ones_hbm, ones_v)
    pltpu.sync_copy(zeros_hbm, acc.at[pl.ds(rbase, ROWS_PT)])
    plsc.subcore_barrier()

    @pl.loop(0, TROWS // DNB)
    def _grp(g0):
        g = g0 * DNB
        for b in range(DNB):
            pltpu.async_copy(ones_v, acc.at[idxd.at[g + b]], sem, add=True)
        for b in range(DNB):
            pltpu.make_async_copy(ones_v, acc.at[idxd.at[g]], sem).wait()

    plsc.subcore_barrier()
    pltpu.sync_copy(
        acc.at[pl.ds(rbase, ROWS_PT)],
        out_hbm.at[pl.ds(cid * NPAD + rbase, ROWS_PT)],
    )


def _prop_body(x_hbm, src_hbm, dst_hbm, zeros_hbm, out_hbm, acc,
               ixs0, ixs1, ixs2, ixs3, ixd0, ixd1, ixd2, ixd3, rows0, rows1,
               semi0, semi1, semi2, semi3, semg0, semg1, sems0, sems1):
    cid = lax.axis_index("c")
    sid = lax.axis_index("s")
    wid = cid * NS + sid
    rbase = sid * ROWS_PT
    ebase = wid * TROWS
    ixs = (ixs0, ixs1, ixs2, ixs3)
    ixd = (ixd0, ixd1, ixd2, ixd3)
    rows = (rows0, rows1)
    semi = (semi0, semi1, semi2, semi3)
    semg = (semg0, semg1)
    sems = (sems0, sems1)

    def idx_issue(j, q):
        pltpu.async_copy(src_hbm.at[pl.ds(ebase + j, 1)], ixs[q], semi[q])
        pltpu.async_copy(dst_hbm.at[pl.ds(ebase + j, 1)], ixd[q], semi[q])

    def idx_wait(q):
        pltpu.make_async_copy(src_hbm.at[pl.ds(0, 1)], ixs[q], semi[q]).wait()
        pltpu.make_async_copy(dst_hbm.at[pl.ds(0, 1)], ixd[q], semi[q]).wait()

    def g_issue(q, b):
        pltpu.async_copy(x_hbm.at[ixs[q].at[0]], rows[b], semg[b])

    def g_wait(q, b):
        pltpu.make_async_copy(x_hbm.at[ixs[q].at[0]], rows[b], semg[b]).wait()

    def s_issue(q, b):
        pltpu.async_copy(rows[b], acc.at[ixd[q].at[0]], sems[b], add=True)

    def s_wait(q, b):
        pltpu.make_async_copy(rows[b], acc.at[ixd[q].at[0]], sems[b]).wait()

    idx_issue(0, 0)
    idx_issue(1, 1)
    idx_issue(2, 2)
    pltpu.sync_copy(zeros_hbm, acc.at[pl.ds(rbase, ROWS_PT)])
    plsc.subcore_barrier()
    idx_wait(0)
    g_issue(0, 0)

    @pl.loop(0, TROWS // 4)
    def _grp(g0):
        base = g0 * 4
        for k in range(4):
            j = base + k
            b = k % 2
            g_wait(k, b)
            s_issue(k, b)

            @pl.when(j >= 1)
            def _wait_prev_scatter():
                s_wait((k + 3) % 4, (k + 1) % 2)

            @pl.when(j + 1 < TROWS)
            def _next_gather():
                idx_wait((k + 1) % 4)
                g_issue((k + 1) % 4, (k + 1) % 2)

            @pl.when(j + 3 < TROWS)
            def _prefetch_idx():
                idx_issue(j + 3, (k + 3) % 4)

    s_wait(3, 1)

    plsc.subcore_barrier()
    pltpu.sync_copy(
        acc.at[pl.ds(rbase, ROWS_PT)],
        out_hbm.at[pl.ds(cid * NPAD + rbase, ROWS_PT)],
    )


_DEG_SCRATCH = [
    pltpu.VMEM_SHARED((NPAD, FEATS), jnp.float32),
    pltpu.VMEM((TROWS, CHUNK), jnp.int32),
    pltpu.VMEM((CHUNK, FEATS), jnp.float32),
    pltpu.SemaphoreType.DMA,
]
_PROP_SCRATCH = (
    [pltpu.VMEM_SHARED((NPAD, FEATS), jnp.float32)]
    + [pltpu.VMEM((1, CHUNK), jnp.int32)] * 8
    + [pltpu.VMEM((CHUNK, FEATS), jnp.float32)] * 2
    + [pltpu.SemaphoreType.DMA] * 8
)

_deg_kernel = pl.kernel(
    _deg_body,
    out_type=jax.ShapeDtypeStruct((NC * NPAD, FEATS), jnp.float32),
    mesh=_mesh,
    scratch_types=_DEG_SCRATCH,
)

_prop_kernel = pl.kernel(
    _prop_body,
    out_type=jax.ShapeDtypeStruct((NC * NPAD, FEATS), jnp.float32),
    mesh=_mesh,
    scratch_types=_PROP_SCRATCH,
)


_RB = 1000


def _deg_of(dp0_ref, dp1_ref):
    return jnp.maximum(dp0_ref[:, 0:1] + dp1_ref[:, 0:1], 1.0)


def _matmul_body(f_ref, w_ref, o_ref):
    o_ref[...] = lax.dot_general(
        f_ref[...], w_ref[...], (((1,), (1,)), ((), ())),
        preferred_element_type=jnp.float32,
        precision=lax.Precision.HIGHEST,
    )


def _scale0_body(dp0_ref, dp1_ref, g_ref, o_ref):
    o_ref[...] = g_ref[...] * lax.rsqrt(_deg_of(dp0_ref, dp1_ref))


def _scale_mid_body(dp0_ref, dp1_ref, r0_ref, r1_ref, o_ref):
    o_ref[...] = (r0_ref[...] + r1_ref[...]) / _deg_of(dp0_ref, dp1_ref)


def _final_body(dp0_ref, dp1_ref, r0_ref, r1_ref, b_ref, o_ref):
    o_ref[...] = ((r0_ref[...] + r1_ref[...])
                  * lax.rsqrt(_deg_of(dp0_ref, dp1_ref)) + b_ref[...])


_row_spec = lambda w: pl.BlockSpec((_RB, w), lambda i: (i, 0))
_full_spec = lambda shape: pl.BlockSpec(shape, lambda i: (0,) * len(shape))
_OUT = jax.ShapeDtypeStruct((N_NODES, FEATS), jnp.float32)

_matmul = pl.pallas_call(
    _matmul_body,
    grid=(N_NODES // _RB,),
    in_specs=[_row_spec(FEATS), _full_spec((FEATS, FEATS))],
    out_specs=_row_spec(FEATS),
    out_shape=_OUT,
)

_scale0 = pl.pallas_call(
    _scale0_body,
    grid=(N_NODES // _RB,),
    in_specs=[_row_spec(FEATS)] * 3,
    out_specs=_row_spec(FEATS),
    out_shape=_OUT,
)

_scale_mid = pl.pallas_call(
    _scale_mid_body,
    grid=(N_NODES // _RB,),
    in_specs=[_row_spec(FEATS)] * 4,
    out_specs=_row_spec(FEATS),
    out_shape=_OUT,
)

_final = pl.pallas_call(
    _final_body,
    grid=(N_NODES // _RB,),
    in_specs=[_row_spec(FEATS)] * 4 + [_full_spec((1, FEATS))],
    out_specs=_row_spec(FEATS),
    out_shape=_OUT,
)


def kernel(feat, edge_index, W, b):
    src = edge_index[0].astype(jnp.int32)
    dst = edge_index[1].astype(jnp.int32)
    npad = EPAD - N_EDGES
    src_p = jnp.concatenate([src, jnp.zeros((npad,), jnp.int32)]).reshape(
        EROWS, CHUNK)
    dst_p = jnp.concatenate([dst, jnp.full((npad,), TRASH, jnp.int32)]).reshape(
        EROWS, CHUNK)
    zeros_rows = jnp.zeros((ROWS_PT, FEATS), jnp.float32)
    ones_rows = jnp.ones((CHUNK, FEATS), jnp.float32)

    g = _matmul(feat, W)
    degp = _deg_kernel(dst_p, ones_rows, zeros_rows)
    dp0, dp1 = degp[:N_NODES], degp[NPAD:NPAD + N_NODES]

    s0 = _scale0(dp0, dp1, g)
    r1 = _prop_kernel(s0, src_p, dst_p, zeros_rows)
    s1 = _scale_mid(dp0, dp1, r1[:N_NODES], r1[NPAD:NPAD + N_NODES])
    r2 = _prop_kernel(s1, src_p, dst_p, zeros_rows)
    out = _final(dp0, dp1, r2[:N_NODES], r2[NPAD:NPAD + N_NODES],
                 b.reshape(1, FEATS))
    return out


# speedup vs baseline: 1.0006x; 1.0006x over previous
import jax
import jax.numpy as jnp
from jax import lax
from jax.experimental import pallas as pl
from jax.experimental.pallas import tpu as pltpu
from jax.experimental.pallas import tpu_sc as plsc

N_NODES = 10000
FEATS = 128
N_EDGES = 320000

NC = 2
NS = 16
NW = NC * NS
CHUNK = 128
EROWS = 2560
TROWS = EROWS // NW
EPAD = EROWS * CHUNK
NPAD = 10112
ROWS_PT = NPAD // NS
TRASH = 10048
NBUF = 4
DNB = 8

_mesh = plsc.VectorSubcoreMesh(core_axis_name="c", subcore_axis_name="s",
                               num_cores=NC, num_subcores=NS)


def _deg_body(dst_hbm, ones_hbm, zeros_hbm, out_hbm, acc, idxd, ones_v, sem):
    cid = lax.axis_index("c")
    sid = lax.axis_index("s")
    wid = cid * NS + sid
    rbase = sid * ROWS_PT

    pltpu.sync_copy(dst_hbm.at[pl.ds(wid * TROWS, TROWS)], idxd)
    pltpu.sync_copy(ones_hbm, ones_v)
    pltpu.sync_copy(zeros_hbm, acc.at[pl.ds(rbase, ROWS_PT)])
    plsc.subcore_barrier()

    @pl.loop(0, TROWS // DNB)
    def _grp(g0):
        g = g0 * DNB
        for b in range(DNB):
            pltpu.async_copy(ones_v, acc.at[idxd.at[g + b]], sem, add=True)
        for b in range(DNB):
            pltpu.make_async_copy(ones_v, acc.at[idxd.at[g]], sem).wait()

    plsc.subcore_barrier()
    pltpu.sync_copy(
        acc.at[pl.ds(rbase, ROWS_PT)],
        out_hbm.at[pl.ds(cid * NPAD + rbase, ROWS_PT)],
    )


def _prop_body(x_hbm, src_hbm, dst_hbm, zeros_hbm, out_hbm, acc,
               ixs0, ixs1, ixs2, ixs3, ixd0, ixd1, ixd2, ixd3, rows0, rows1,
               semi0, semi1, semi2, semi3, semg0, semg1, sems0, sems1):
    cid = lax.axis_index("c")
    sid = lax.axis_index("s")
    wid = cid * NS + sid
    rbase = sid * ROWS_PT
    ebase = wid * TROWS
    ixs = (ixs0, ixs1, ixs2, ixs3)
    ixd = (ixd0, ixd1, ixd2, ixd3)
    rows = (rows0, rows1)
    semi = (semi0, semi1, semi2, semi3)
    semg = (semg0, semg1)
    sems = (sems0, sems1)

    def idx_issue(j, q):
        pltpu.async_copy(src_hbm.at[pl.ds(ebase + j, 1)], ixs[q], semi[q])
        pltpu.async_copy(dst_hbm.at[pl.ds(ebase + j, 1)], ixd[q], semi[q])

    def idx_wait(q):
        pltpu.make_async_copy(src_hbm.at[pl.ds(0, 1)], ixs[q], semi[q]).wait()
        pltpu.make_async_copy(dst_hbm.at[pl.ds(0, 1)], ixd[q], semi[q]).wait()

    def g_issue(q, b):
        pltpu.async_copy(x_hbm.at[ixs[q].at[0]], rows[b], semg[b])

    def g_wait(q, b):
        pltpu.make_async_copy(x_hbm.at[ixs[q].at[0]], rows[b], semg[b]).wait()

    def s_issue(q, b):
        pltpu.async_copy(rows[b], acc.at[ixd[q].at[0]], sems[b], add=True)

    def s_wait(q, b):
        pltpu.make_async_copy(rows[b], acc.at[ixd[q].at[0]], sems[b]).wait()

    idx_issue(0, 0)
    idx_issue(1, 1)
    idx_issue(2, 2)
    pltpu.sync_copy(zeros_hbm, acc.at[pl.ds(rbase, ROWS_PT)])
    plsc.subcore_barrier()
    idx_wait(0)
    g_issue(0, 0)

    @pl.loop(0, TROWS // 4)
    def _grp(g0):
        base = g0 * 4
        for k in range(4):
            j = base + k
            b = k % 2
            g_wait(k, b)
            s_issue(k, b)

            @pl.when(j >= 1)
            def _wait_prev_scatter():
                s_wait((k + 3) % 4, (k + 1) % 2)

            @pl.when(j + 1 < TROWS)
            def _next_gather():
                idx_wait((k + 1) % 4)
                g_issue((k + 1) % 4, (k + 1) % 2)

            @pl.when(j + 3 < TROWS)
            def _prefetch_idx():
                idx_issue(j + 3, (k + 3) % 4)

    s_wait(3, 1)

    plsc.subcore_barrier()
    pltpu.sync_copy(
        acc.at[pl.ds(rbase, ROWS_PT)],
        out_hbm.at[pl.ds(cid * NPAD + rbase, ROWS_PT)],
    )


_DEG_SCRATCH = [
    pltpu.VMEM_SHARED((NPAD, FEATS), jnp.float32),
    pltpu.VMEM((TROWS, CHUNK), jnp.int32),
    pltpu.VMEM((CHUNK, FEATS), jnp.float32),
    pltpu.SemaphoreType.DMA,
]
_PROP_SCRATCH = (
    [pltpu.VMEM_SHARED((NPAD, FEATS), jnp.float32)]
    + [pltpu.VMEM((1, CHUNK), jnp.int32)] * 8
    + [pltpu.VMEM((CHUNK, FEATS), jnp.float32)] * 2
    + [pltpu.SemaphoreType.DMA] * 8
)

_deg_kernel = pl.kernel(
    _deg_body,
    out_type=jax.ShapeDtypeStruct((NC * NPAD, FEATS), jnp.float32),
    mesh=_mesh,
    scratch_types=_DEG_SCRATCH,
)

_prop_kernel = pl.kernel(
    _prop_body,
    out_type=jax.ShapeDtypeStruct((NC * NPAD, FEATS), jnp.float32),
    mesh=_mesh,
    scratch_types=_PROP_SCRATCH,
)


_RB = 1000


def _deg_of(dp0_ref, dp1_ref):
    return jnp.maximum(dp0_ref[:, 0:1] + dp1_ref[:, 0:1], 1.0)


def _matmul_body(f_ref, w_ref, o_ref):
    o_ref[...] = lax.dot_general(
        f_ref[...], w_ref[...], (((1,), (1,)), ((), ())),
        preferred_element_type=jnp.float32,
        precision=lax.Precision.HIGHEST,
    )


def _scale0_body(dp0_ref, dp1_ref, g_ref, o_ref):
    o_ref[...] = g_ref[...] * lax.rsqrt(_deg_of(dp0_ref, dp1_ref))


def _scale_mid_body(dp0_ref, dp1_ref, r0_ref, r1_ref, o_ref):
    o_ref[...] = (r0_ref[...] + r1_ref[...]) / _deg_of(dp0_ref, dp1_ref)


def _final_body(dp0_ref, dp1_ref, r0_ref, r1_ref, b_ref, o_ref):
    o_ref[...] = ((r0_ref[...] + r1_ref[...])
                  * lax.rsqrt(_deg_of(dp0_ref, dp1_ref)) + b_ref[...])


_row_spec = lambda w: pl.BlockSpec((_RB, w), lambda i: (i, 0))
_full_spec = lambda shape: pl.BlockSpec(shape, lambda i: (0,) * len(shape))
_OUT = jax.ShapeDtypeStruct((N_NODES, FEATS), jnp.float32)

_matmul = pl.pallas_call(
    _matmul_body,
    grid=(N_NODES // _RB,),
    in_specs=[_row_spec(FEATS), _full_spec((FEATS, FEATS))],
    out_specs=_row_spec(FEATS),
    out_shape=_OUT,
)

_scale0 = pl.pallas_call(
    _scale0_body,
    grid=(N_NODES // _RB,),
    in_specs=[_row_spec(FEATS)] * 3,
    out_specs=_row_spec(FEATS),
    out_shape=_OUT,
)

_scale_mid = pl.pallas_call(
    _scale_mid_body,
    grid=(N_NODES // _RB,),
    in_specs=[_row_spec(FEATS)] * 4,
    out_specs=_row_spec(FEATS),
    out_shape=_OUT,
)

_final = pl.pallas_call(
    _final_body,
    grid=(N_NODES // _RB,),
    in_specs=[_row_spec(FEATS)] * 4 + [_full_spec((1, FEATS))],
    out_specs=_row_spec(FEATS),
    out_shape=_OUT,
)


def kernel(feat, edge_index, W, b):
    src = edge_index[0].astype(jnp.int32)
    dst = edge_index[1].astype(jnp.int32)
    npad = EPAD - N_EDGES
    src_p = jnp.concatenate([src, jnp.zeros((npad,), jnp.int32)]).reshape(
        EROWS, CHUNK)
    trash = TRASH + jnp.arange(npad, dtype=jnp.int32) % (NPAD - TRASH)
    dst_p = jnp.concatenate([dst, trash]).reshape(EROWS, CHUNK)
    zeros_rows = jnp.zeros((ROWS_PT, FEATS), jnp.float32)
    ones_rows = jnp.ones((CHUNK, FEATS), jnp.float32)

    g = _matmul(feat, W)
    degp = _deg_kernel(dst_p, ones_rows, zeros_rows)
    dp0, dp1 = degp[:N_NODES], degp[NPAD:NPAD + N_NODES]

    s0 = _scale0(dp0, dp1, g)
    r1 = _prop_kernel(s0, src_p, dst_p, zeros_rows)
    s1 = _scale_mid(dp0, dp1, r1[:N_NODES], r1[NPAD:NPAD + N_NODES])
    r2 = _prop_kernel(s1, src_p, dst_p, zeros_rows)
    out = _final(dp0, dp1, r2[:N_NODES], r2[NPAD:NPAD + N_NODES],
                 b.reshape(1, FEATS))
    return out


# speedup vs baseline: 2.8071x; 2.8054x over previous
import jax
import jax.numpy as jnp
from jax import lax
from jax.experimental import pallas as pl
from jax.experimental.pallas import tpu as pltpu
from jax.experimental.pallas import tpu_sc as plsc

N_NODES = 10000
FEATS = 128
N_EDGES = 320000

NC = 2
NS = 16
NW = NC * NS
CHUNK = 128
EROWS = 2560
TROWS = EROWS // NW
EPAD = EROWS * CHUNK
NPAD = 10112
ROWS_PT = NPAD // NS
TRASH = 10048
NBUF = 4
DNB = 8

_mesh = plsc.VectorSubcoreMesh(core_axis_name="c", subcore_axis_name="s",
                               num_cores=NC, num_subcores=NS)


def _deg_body(dst_hbm, ones_hbm, zeros_hbm, out_hbm, acc, idxd, ones_v, sem):
    cid = lax.axis_index("c")
    sid = lax.axis_index("s")
    wid = cid * NS + sid
    rbase = sid * ROWS_PT

    pltpu.sync_copy(dst_hbm.at[pl.ds(wid * TROWS, TROWS)], idxd)
    pltpu.sync_copy(ones_hbm, ones_v)
    pltpu.sync_copy(zeros_hbm, acc.at[pl.ds(rbase, ROWS_PT)])
    plsc.subcore_barrier()

    @pl.loop(0, TROWS // DNB)
    def _grp(g0):
        g = g0 * DNB
        for b in range(DNB):
            pltpu.async_copy(ones_v, acc.at[idxd.at[g + b]], sem, add=True)
        for b in range(DNB):
            pltpu.make_async_copy(ones_v, acc.at[idxd.at[g]], sem).wait()

    plsc.subcore_barrier()
    pltpu.sync_copy(
        acc.at[pl.ds(rbase, ROWS_PT)],
        out_hbm.at[pl.ds(cid * NPAD + rbase, ROWS_PT)],
    )


def _prop_body(x_hbm, src_hbm, dst_hbm, zeros_hbm, out_hbm, acc,
               ixs0, ixs1, ixs2, ixs3, ixd0, ixd1, ixd2, ixd3, rows0, rows1,
               semi0, semi1, semi2, semi3, semg0, semg1, sems0, sems1):
    cid = lax.axis_index("c")
    sid = lax.axis_index("s")
    wid = cid * NS + sid
    rbase = sid * ROWS_PT
    ebase = wid * TROWS
    ixs = (ixs0, ixs1, ixs2, ixs3)
    ixd = (ixd0, ixd1, ixd2, ixd3)
    rows = (rows0, rows1)
    semi = (semi0, semi1, semi2, semi3)
    semg = (semg0, semg1)
    sems = (sems0, sems1)

    def idx_issue(j, q):
        pltpu.async_copy(src_hbm.at[pl.ds(ebase + j, 1)], ixs[q], semi[q])
        pltpu.async_copy(dst_hbm.at[pl.ds(ebase + j, 1)], ixd[q], semi[q])

    def idx_wait(q):
        pltpu.make_async_copy(src_hbm.at[pl.ds(0, 1)], ixs[q], semi[q]).wait()
        pltpu.make_async_copy(dst_hbm.at[pl.ds(0, 1)], ixd[q], semi[q]).wait()

    def g_issue(q, b):
        pltpu.async_copy(x_hbm.at[ixs[q].at[0]], rows[b], semg[b])

    def g_wait(q, b):
        pltpu.make_async_copy(x_hbm.at[ixs[q].at[0]], rows[b], semg[b]).wait()

    def s_issue(q, b):
        pltpu.async_copy(rows[b], acc.at[ixd[q].at[0]], sems[b], add=True)

    def s_wait(q, b):
        pltpu.make_async_copy(rows[b], acc.at[ixd[q].at[0]], sems[b]).wait()

    idx_issue(0, 0)
    idx_issue(1, 1)
    idx_issue(2, 2)
    pltpu.sync_copy(zeros_hbm, acc.at[pl.ds(rbase, ROWS_PT)])
    plsc.subcore_barrier()
    idx_wait(0)
    g_issue(0, 0)

    @pl.loop(0, TROWS // 4)
    def _grp(g0):
        base = g0 * 4
        for k in range(4):
            j = base + k
            b = k % 2
            g_wait(k, b)
            s_issue(k, b)

            @pl.when(j >= 1)
            def _wait_prev_scatter():
                s_wait((k + 3) % 4, (k + 1) % 2)

            @pl.when(j + 1 < TROWS)
            def _next_gather():
                idx_wait((k + 1) % 4)
                g_issue((k + 1) % 4, (k + 1) % 2)

            @pl.when(j + 3 < TROWS)
            def _prefetch_idx():
                idx_issue(j + 3, (k + 3) % 4)

    s_wait(3, 1)

    plsc.subcore_barrier()
    pltpu.sync_copy(
        acc.at[pl.ds(rbase, ROWS_PT)],
        out_hbm.at[pl.ds(cid * NPAD + rbase, ROWS_PT)],
    )


_DEG_SCRATCH = [
    pltpu.VMEM_SHARED((NPAD, FEATS), jnp.float32),
    pltpu.VMEM((TROWS, CHUNK), jnp.int32),
    pltpu.VMEM((CHUNK, FEATS), jnp.float32),
    pltpu.SemaphoreType.DMA,
]
_PROP_SCRATCH = (
    [pltpu.VMEM_SHARED((NPAD, FEATS), jnp.float32)]
    + [pltpu.VMEM((1, CHUNK), jnp.int32)] * 8
    + [pltpu.VMEM((CHUNK, FEATS), jnp.float32)] * 2
    + [pltpu.SemaphoreType.DMA] * 8
)

_deg_kernel = pl.kernel(
    _deg_body,
    out_type=jax.ShapeDtypeStruct((NC * NPAD, FEATS), jnp.float32),
    mesh=_mesh,
    scratch_types=_DEG_SCRATCH,
)

_prop_kernel = pl.kernel(
    _prop_body,
    out_type=jax.ShapeDtypeStruct((NC * NPAD, FEATS), jnp.float32),
    mesh=_mesh,
    scratch_types=_PROP_SCRATCH,
)


_RB = 1000


def _deg_of(dp0_ref, dp1_ref):
    return jnp.maximum(dp0_ref[:, 0:1] + dp1_ref[:, 0:1], 1.0)


def _matmul_body(f_ref, w_ref, o_ref):
    o_ref[...] = lax.dot_general(
        f_ref[...], w_ref[...], (((1,), (1,)), ((), ())),
        preferred_element_type=jnp.float32,
        precision=lax.Precision.HIGHEST,
    )


def _scale0_body(dp0_ref, dp1_ref, g_ref, o_ref):
    o_ref[...] = g_ref[...] * lax.rsqrt(_deg_of(dp0_ref, dp1_ref))


def _scale_mid_body(dp0_ref, dp1_ref, r0_ref, r1_ref, o_ref):
    o_ref[...] = (r0_ref[...] + r1_ref[...]) / _deg_of(dp0_ref, dp1_ref)


def _final_body(dp0_ref, dp1_ref, r0_ref, r1_ref, b_ref, o_ref):
    o_ref[...] = ((r0_ref[...] + r1_ref[...])
                  * lax.rsqrt(_deg_of(dp0_ref, dp1_ref)) + b_ref[...])


_row_spec = lambda w: pl.BlockSpec((_RB, w), lambda i: (i, 0))
_full_spec = lambda shape: pl.BlockSpec(shape, lambda i: (0,) * len(shape))
_OUT = jax.ShapeDtypeStruct((N_NODES, FEATS), jnp.float32)

_matmul = pl.pallas_call(
    _matmul_body,
    grid=(N_NODES // _RB,),
    in_specs=[_row_spec(FEATS), _full_spec((FEATS, FEATS))],
    out_specs=_row_spec(FEATS),
    out_shape=_OUT,
)

_scale0 = pl.pallas_call(
    _scale0_body,
    grid=(N_NODES // _RB,),
    in_specs=[_row_spec(FEATS)] * 3,
    out_specs=_row_spec(FEATS),
    out_shape=_OUT,
)

_scale_mid = pl.pallas_call(
    _scale_mid_body,
    grid=(N_NODES // _RB,),
    in_specs=[_row_spec(FEATS)] * 4,
    out_specs=_row_spec(FEATS),
    out_shape=_OUT,
)

_final = pl.pallas_call(
    _final_body,
    grid=(N_NODES // _RB,),
    in_specs=[_row_spec(FEATS)] * 4 + [_full_spec((1, FEATS))],
    out_specs=_row_spec(FEATS),
    out_shape=_OUT,
)


def kernel(feat, edge_index, W, b):
    src = edge_index[0].astype(jnp.int32)
    dst = edge_index[1].astype(jnp.int32)
    npad = EPAD - N_EDGES
    pad_src = jnp.arange(npad, dtype=jnp.int32) % N_NODES
    src_p = jnp.concatenate([src, pad_src]).reshape(EROWS, CHUNK)
    trash = TRASH + jnp.arange(npad, dtype=jnp.int32) % (NPAD - TRASH)
    dst_p = jnp.concatenate([dst, trash]).reshape(EROWS, CHUNK)
    zeros_rows = jnp.zeros((ROWS_PT, FEATS), jnp.float32)
    ones_rows = jnp.ones((CHUNK, FEATS), jnp.float32)

    g = _matmul(feat, W)
    degp = _deg_kernel(dst_p, ones_rows, zeros_rows)
    dp0, dp1 = degp[:N_NODES], degp[NPAD:NPAD + N_NODES]

    s0 = _scale0(dp0, dp1, g)
    r1 = _prop_kernel(s0, src_p, dst_p, zeros_rows)
    s1 = _scale_mid(dp0, dp1, r1[:N_NODES], r1[NPAD:NPAD + N_NODES])
    r2 = _prop_kernel(s1, src_p, dst_p, zeros_rows)
    out = _final(dp0, dp1, r2[:N_NODES], r2[NPAD:NPAD + N_NODES],
                 b.reshape(1, FEATS))
    return out


# speedup vs baseline: 3.1586x; 1.1252x over previous
import jax
import jax.numpy as jnp
from jax import lax
from jax.experimental import pallas as pl
from jax.experimental.pallas import tpu as pltpu
from jax.experimental.pallas import tpu_sc as plsc

N_NODES = 10000
FEATS = 128
N_EDGES = 320000

NC = 2
NS = 16
NW = NC * NS
CHUNK = 128
EROWS = 2560
TROWS = EROWS // NW
EPAD = EROWS * CHUNK
NPAD = 10112
ROWS_PT = NPAD // NS
TRASH = 10048
NBUF = 4
DNB = 8

_mesh = plsc.VectorSubcoreMesh(core_axis_name="c", subcore_axis_name="s",
                               num_cores=NC, num_subcores=NS)


def _deg_body(dst_hbm, ones_hbm, zeros_hbm, out_hbm, acc, idxd, ones_v, sem):
    cid = lax.axis_index("c")
    sid = lax.axis_index("s")
    wid = cid * NS + sid
    rbase = sid * ROWS_PT

    pltpu.sync_copy(dst_hbm.at[pl.ds(wid * TROWS, TROWS)], idxd)
    pltpu.sync_copy(ones_hbm, ones_v)
    pltpu.sync_copy(zeros_hbm, acc.at[pl.ds(rbase, ROWS_PT)])
    plsc.subcore_barrier()

    @pl.loop(0, TROWS // DNB)
    def _grp(g0):
        g = g0 * DNB
        for b in range(DNB):
            pltpu.async_copy(ones_v, acc.at[idxd.at[g + b]], sem, add=True)
        for b in range(DNB):
            pltpu.make_async_copy(ones_v, acc.at[idxd.at[g]], sem).wait()

    plsc.subcore_barrier()
    pltpu.sync_copy(
        acc.at[pl.ds(rbase, ROWS_PT)],
        out_hbm.at[pl.ds(cid * NPAD + rbase, ROWS_PT)],
    )


def _prop_body(x_hbm, src_hbm, dst_hbm, zeros_hbm, out_hbm, acc,
               ixs0, ixs1, ixs2, ixs3, ixd0, ixd1, ixd2, ixd3, rows0, rows1,
               semi0, semi1, semi2, semi3, semg0, semg1, sems0, sems1):
    cid = lax.axis_index("c")
    sid = lax.axis_index("s")
    wid = cid * NS + sid
    rbase = sid * ROWS_PT
    ebase = wid * TROWS
    ixs = (ixs0, ixs1, ixs2, ixs3)
    ixd = (ixd0, ixd1, ixd2, ixd3)
    rows = (rows0, rows1)
    semi = (semi0, semi1, semi2, semi3)
    semg = (semg0, semg1)
    sems = (sems0, sems1)

    def idx_issue(j, q):
        pltpu.async_copy(src_hbm.at[pl.ds(ebase + j, 1)], ixs[q], semi[q])
        pltpu.async_copy(dst_hbm.at[pl.ds(ebase + j, 1)], ixd[q], semi[q])

    def idx_wait(q):
        pltpu.make_async_copy(src_hbm.at[pl.ds(0, 1)], ixs[q], semi[q]).wait()
        pltpu.make_async_copy(dst_hbm.at[pl.ds(0, 1)], ixd[q], semi[q]).wait()

    def g_issue(q, b):
        pltpu.async_copy(x_hbm.at[ixs[q].at[0]], rows[b], semg[b])

    def g_wait(q, b):
        pltpu.make_async_copy(x_hbm.at[ixs[q].at[0]], rows[b], semg[b]).wait()

    def s_issue(q, b):
        pltpu.async_copy(rows[b], acc.at[ixd[q].at[0]], sems[b], add=True)

    def s_wait(q, b):
        pltpu.make_async_copy(rows[b], acc.at[ixd[q].at[0]], sems[b]).wait()

    idx_issue(0, 0)
    idx_issue(1, 1)
    idx_issue(2, 2)
    pltpu.sync_copy(zeros_hbm, acc.at[pl.ds(rbase, ROWS_PT)])
    plsc.subcore_barrier()
    idx_wait(0)
    g_issue(0, 0)

    @pl.loop(0, TROWS // 4)
    def _grp(g0):
        base = g0 * 4
        for k in range(4):
            j = base + k
            b = k % 2

            @pl.when(j >= 1)
            def _wait_prev_scatter():
                s_wait((k + 3) % 4, (k + 1) % 2)

            @pl.when(j + 1 < TROWS)
            def _next_gather():
                idx_wait((k + 1) % 4)
                g_issue((k + 1) % 4, (k + 1) % 2)

            g_wait(k, b)
            s_issue(k, b)

            @pl.when(j + 3 < TROWS)
            def _prefetch_idx():
                idx_issue(j + 3, (k + 3) % 4)

    s_wait(3, 1)

    plsc.subcore_barrier()
    pltpu.sync_copy(
        acc.at[pl.ds(rbase, ROWS_PT)],
        out_hbm.at[pl.ds(cid * NPAD + rbase, ROWS_PT)],
    )


_DEG_SCRATCH = [
    pltpu.VMEM_SHARED((NPAD, FEATS), jnp.float32),
    pltpu.VMEM((TROWS, CHUNK), jnp.int32),
    pltpu.VMEM((CHUNK, FEATS), jnp.float32),
    pltpu.SemaphoreType.DMA,
]
_PROP_SCRATCH = (
    [pltpu.VMEM_SHARED((NPAD, FEATS), jnp.float32)]
    + [pltpu.VMEM((1, CHUNK), jnp.int32)] * 8
    + [pltpu.VMEM((CHUNK, FEATS), jnp.float32)] * 2
    + [pltpu.SemaphoreType.DMA] * 8
)

_deg_kernel = pl.kernel(
    _deg_body,
    out_type=jax.ShapeDtypeStruct((NC * NPAD, FEATS), jnp.float32),
    mesh=_mesh,
    scratch_types=_DEG_SCRATCH,
)

_prop_kernel = pl.kernel(
    _prop_body,
    out_type=jax.ShapeDtypeStruct((NC * NPAD, FEATS), jnp.float32),
    mesh=_mesh,
    scratch_types=_PROP_SCRATCH,
)


_RB = 1000


def _deg_of(dp0_ref, dp1_ref):
    return jnp.maximum(dp0_ref[:, 0:1] + dp1_ref[:, 0:1], 1.0)


def _matmul_body(f_ref, w_ref, o_ref):
    o_ref[...] = lax.dot_general(
        f_ref[...], w_ref[...], (((1,), (1,)), ((), ())),
        preferred_element_type=jnp.float32,
        precision=lax.Precision.HIGHEST,
    )


def _scale0_body(dp0_ref, dp1_ref, g_ref, o_ref):
    o_ref[...] = g_ref[...] * lax.rsqrt(_deg_of(dp0_ref, dp1_ref))


def _scale_mid_body(dp0_ref, dp1_ref, r0_ref, r1_ref, o_ref):
    o_ref[...] = (r0_ref[...] + r1_ref[...]) / _deg_of(dp0_ref, dp1_ref)


def _final_body(dp0_ref, dp1_ref, r0_ref, r1_ref, b_ref, o_ref):
    o_ref[...] = ((r0_ref[...] + r1_ref[...])
                  * lax.rsqrt(_deg_of(dp0_ref, dp1_ref)) + b_ref[...])


_row_spec = lambda w: pl.BlockSpec((_RB, w), lambda i: (i, 0))
_full_spec = lambda shape: pl.BlockSpec(shape, lambda i: (0,) * len(shape))
_OUT = jax.ShapeDtypeStruct((N_NODES, FEATS), jnp.float32)

_matmul = pl.pallas_call(
    _matmul_body,
    grid=(N_NODES // _RB,),
    in_specs=[_row_spec(FEATS), _full_spec((FEATS, FEATS))],
    out_specs=_row_spec(FEATS),
    out_shape=_OUT,
)

_scale0 = pl.pallas_call(
    _scale0_body,
    grid=(N_NODES // _RB,),
    in_specs=[_row_spec(FEATS)] * 3,
    out_specs=_row_spec(FEATS),
    out_shape=_OUT,
)

_scale_mid = pl.pallas_call(
    _scale_mid_body,
    grid=(N_NODES // _RB,),
    in_specs=[_row_spec(FEATS)] * 4,
    out_specs=_row_spec(FEATS),
    out_shape=_OUT,
)

_final = pl.pallas_call(
    _final_body,
    grid=(N_NODES // _RB,),
    in_specs=[_row_spec(FEATS)] * 4 + [_full_spec((1, FEATS))],
    out_specs=_row_spec(FEATS),
    out_shape=_OUT,
)


def kernel(feat, edge_index, W, b):
    src = edge_index[0].astype(jnp.int32)
    dst = edge_index[1].astype(jnp.int32)
    npad = EPAD - N_EDGES
    pad_src = jnp.arange(npad, dtype=jnp.int32) % N_NODES
    src_p = jnp.concatenate([src, pad_src]).reshape(EROWS, CHUNK)
    trash = TRASH + jnp.arange(npad, dtype=jnp.int32) % (NPAD - TRASH)
    dst_p = jnp.concatenate([dst, trash]).reshape(EROWS, CHUNK)
    zeros_rows = jnp.zeros((ROWS_PT, FEATS), jnp.float32)
    ones_rows = jnp.ones((CHUNK, FEATS), jnp.float32)

    g = _matmul(feat, W)
    degp = _deg_kernel(dst_p, ones_rows, zeros_rows)
    dp0, dp1 = degp[:N_NODES], degp[NPAD:NPAD + N_NODES]

    s0 = _scale0(dp0, dp1, g)
    r1 = _prop_kernel(s0, src_p, dst_p, zeros_rows)
    s1 = _scale_mid(dp0, dp1, r1[:N_NODES], r1[NPAD:NPAD + N_NODES])
    r2 = _prop_kernel(s1, src_p, dst_p, zeros_rows)
    out = _final(dp0, dp1, r2[:N_NODES], r2[NPAD:NPAD + N_NODES],
                 b.reshape(1, FEATS))
    return out


# speedup vs baseline: 3.3342x; 1.0556x over previous
import jax
import jax.numpy as jnp
from jax import lax
from jax.experimental import pallas as pl
from jax.experimental.pallas import tpu as pltpu
from jax.experimental.pallas import tpu_sc as plsc

N_NODES = 10000
FEATS = 128
N_EDGES = 320000

NC = 2
NS = 16
NW = NC * NS
CHUNK = 128
EROWS = 2560
TROWS = EROWS // NW
EPAD = EROWS * CHUNK
ACC_ROWS = 10064
ROWS_PT = 632
LAST_PT = N_NODES - 15 * ROWS_PT
TRASH = 10000
NBUF = 4
DNB = 8

_mesh = plsc.VectorSubcoreMesh(core_axis_name="c", subcore_axis_name="s",
                               num_cores=NC, num_subcores=NS)


def _zero_my_rows(zeros_hbm, acc, sid, rbase):
    @pl.when(sid < 15)
    def _():
        pltpu.sync_copy(zeros_hbm, acc.at[pl.ds(rbase, ROWS_PT)])

    @pl.when(sid == 15)
    def _():
        pltpu.sync_copy(zeros_hbm.at[pl.ds(0, ACC_ROWS - 15 * ROWS_PT)],
                        acc.at[pl.ds(15 * ROWS_PT, ACC_ROWS - 15 * ROWS_PT)])


def _writeback_my_rows(acc, out_hbm, cid, sid, rbase):
    @pl.when(sid < 15)
    def _():
        pltpu.sync_copy(
            acc.at[pl.ds(rbase, ROWS_PT)],
            out_hbm.at[pl.ds(cid * N_NODES + rbase, ROWS_PT)],
        )

    @pl.when(sid == 15)
    def _():
        pltpu.sync_copy(
            acc.at[pl.ds(15 * ROWS_PT, LAST_PT)],
            out_hbm.at[pl.ds(cid * N_NODES + 15 * ROWS_PT, LAST_PT)],
        )


def _deg_body(dst_hbm, ones_hbm, zeros_hbm, out_hbm, acc, idxd, ones_v, sem):
    cid = lax.axis_index("c")
    sid = lax.axis_index("s")
    wid = cid * NS + sid
    rbase = sid * ROWS_PT

    pltpu.sync_copy(dst_hbm.at[pl.ds(wid * TROWS, TROWS)], idxd)
    pltpu.sync_copy(ones_hbm, ones_v)
    _zero_my_rows(zeros_hbm, acc, sid, rbase)
    plsc.subcore_barrier()

    @pl.loop(0, TROWS // DNB)
    def _grp(g0):
        g = g0 * DNB
        for b in range(DNB):
            pltpu.async_copy(ones_v, acc.at[idxd.at[g + b]], sem, add=True)
        for b in range(DNB):
            pltpu.make_async_copy(ones_v, acc.at[idxd.at[g]], sem).wait()

    plsc.subcore_barrier()
    _writeback_my_rows(acc, out_hbm, cid, sid, rbase)


def _prop_body(x_hbm, src_hbm, dst_hbm, zeros_hbm, out_hbm, acc,
               ixs0, ixs1, ixs2, ixs3, ixd0, ixd1, ixd2, ixd3, rows0, rows1,
               semi0, semi1, semi2, semi3, semg0, semg1, sems0, sems1):
    cid = lax.axis_index("c")
    sid = lax.axis_index("s")
    wid = cid * NS + sid
    rbase = sid * ROWS_PT
    ebase = wid * TROWS
    ixs = (ixs0, ixs1, ixs2, ixs3)
    ixd = (ixd0, ixd1, ixd2, ixd3)
    rows = (rows0, rows1)
    semi = (semi0, semi1, semi2, semi3)
    semg = (semg0, semg1)
    sems = (sems0, sems1)

    def idx_issue(j, q):
        pltpu.async_copy(src_hbm.at[pl.ds(ebase + j, 1)], ixs[q], semi[q])
        pltpu.async_copy(dst_hbm.at[pl.ds(ebase + j, 1)], ixd[q], semi[q])

    def idx_wait(q):
        pltpu.make_async_copy(src_hbm.at[pl.ds(0, 1)], ixs[q], semi[q]).wait()
        pltpu.make_async_copy(dst_hbm.at[pl.ds(0, 1)], ixd[q], semi[q]).wait()

    def g_issue(q, b):
        pltpu.async_copy(x_hbm.at[ixs[q].at[0]], rows[b], semg[b])

    def g_wait(q, b):
        pltpu.make_async_copy(x_hbm.at[ixs[q].at[0]], rows[b], semg[b]).wait()

    def s_issue(q, b):
        pltpu.async_copy(rows[b], acc.at[ixd[q].at[0]], sems[b], add=True)

    def s_wait(q, b):
        pltpu.make_async_copy(rows[b], acc.at[ixd[q].at[0]], sems[b]).wait()

    idx_issue(0, 0)
    idx_issue(1, 1)
    idx_issue(2, 2)
    _zero_my_rows(zeros_hbm, acc, sid, rbase)
    plsc.subcore_barrier()
    idx_wait(0)
    g_issue(0, 0)

    @pl.loop(0, TROWS // 4)
    def _grp(g0):
        base = g0 * 4
        for k in range(4):
            j = base + k
            b = k % 2

            @pl.when(j >= 1)
            def _wait_prev_scatter():
                s_wait((k + 3) % 4, (k + 1) % 2)

            @pl.when(j + 1 < TROWS)
            def _next_gather():
                idx_wait((k + 1) % 4)
                g_issue((k + 1) % 4, (k + 1) % 2)

            g_wait(k, b)
            s_issue(k, b)

            @pl.when(j + 3 < TROWS)
            def _prefetch_idx():
                idx_issue(j + 3, (k + 3) % 4)

    s_wait(3, 1)

    plsc.subcore_barrier()
    _writeback_my_rows(acc, out_hbm, cid, sid, rbase)


_DEG_SCRATCH = [
    pltpu.VMEM_SHARED((ACC_ROWS, FEATS), jnp.float32),
    pltpu.VMEM((TROWS, CHUNK), jnp.int32),
    pltpu.VMEM((CHUNK, FEATS), jnp.float32),
    pltpu.SemaphoreType.DMA,
]
_PROP_SCRATCH = (
    [pltpu.VMEM_SHARED((ACC_ROWS, FEATS), jnp.float32)]
    + [pltpu.VMEM((1, CHUNK), jnp.int32)] * 8
    + [pltpu.VMEM((CHUNK, FEATS), jnp.float32)] * 2
    + [pltpu.SemaphoreType.DMA] * 8
)

_deg_kernel = pl.kernel(
    _deg_body,
    out_type=jax.ShapeDtypeStruct((NC * N_NODES, FEATS), jnp.float32),
    mesh=_mesh,
    scratch_types=_DEG_SCRATCH,
)

_prop_kernel = pl.kernel(
    _prop_body,
    out_type=jax.ShapeDtypeStruct((NC * N_NODES, FEATS), jnp.float32),
    mesh=_mesh,
    scratch_types=_PROP_SCRATCH,
)


_RB = 1000


def _deg_of(dp0_ref, dp1_ref):
    return jnp.maximum(dp0_ref[:, 0:1] + dp1_ref[:, 0:1], 1.0)


def _matmul_body(f_ref, w_ref, o_ref):
    o_ref[...] = lax.dot_general(
        f_ref[...], w_ref[...], (((1,), (1,)), ((), ())),
        preferred_element_type=jnp.float32,
        precision=lax.Precision.HIGHEST,
    )


def _scale0_body(dp0_ref, dp1_ref, g_ref, o_ref):
    o_ref[...] = g_ref[...] * lax.rsqrt(_deg_of(dp0_ref, dp1_ref))


def _scale_mid_body(dp0_ref, dp1_ref, r0_ref, r1_ref, o_ref):
    o_ref[...] = (r0_ref[...] + r1_ref[...]) / _deg_of(dp0_ref, dp1_ref)


def _final_body(dp0_ref, dp1_ref, r0_ref, r1_ref, b_ref, o_ref):
    o_ref[...] = ((r0_ref[...] + r1_ref[...])
                  * lax.rsqrt(_deg_of(dp0_ref, dp1_ref)) + b_ref[...])


_row_spec = lambda w: pl.BlockSpec((_RB, w), lambda i: (i, 0))
_half2_spec = pl.BlockSpec((_RB, FEATS), lambda i: (i + N_NODES // _RB, 0))
_full_spec = lambda shape: pl.BlockSpec(shape, lambda i: (0,) * len(shape))
_OUT = jax.ShapeDtypeStruct((N_NODES, FEATS), jnp.float32)

_matmul = pl.pallas_call(
    _matmul_body,
    grid=(N_NODES // _RB,),
    in_specs=[_row_spec(FEATS), _full_spec((FEATS, FEATS))],
    out_specs=_row_spec(FEATS),
    out_shape=_OUT,
)

_scale0 = pl.pallas_call(
    _scale0_body,
    grid=(N_NODES // _RB,),
    in_specs=[_row_spec(FEATS), _half2_spec, _row_spec(FEATS)],
    out_specs=_row_spec(FEATS),
    out_shape=_OUT,
)

_scale_mid = pl.pallas_call(
    _scale_mid_body,
    grid=(N_NODES // _RB,),
    in_specs=[_row_spec(FEATS), _half2_spec, _row_spec(FEATS), _half2_spec],
    out_specs=_row_spec(FEATS),
    out_shape=_OUT,
)

_final = pl.pallas_call(
    _final_body,
    grid=(N_NODES // _RB,),
    in_specs=[_row_spec(FEATS), _half2_spec, _row_spec(FEATS), _half2_spec,
              _full_spec((1, FEATS))],
    out_specs=_row_spec(FEATS),
    out_shape=_OUT,
)


def kernel(feat, edge_index, W, b):
    src = edge_index[0].astype(jnp.int32)
    dst = edge_index[1].astype(jnp.int32)
    npad = EPAD - N_EDGES
    pad_src = jnp.arange(npad, dtype=jnp.int32) % N_NODES
    src_p = jnp.concatenate([src, pad_src]).reshape(EROWS, CHUNK)
    trash = TRASH + jnp.arange(npad, dtype=jnp.int32) % (ACC_ROWS - TRASH)
    dst_p = jnp.concatenate([dst, trash]).reshape(EROWS, CHUNK)
    zeros_rows = jnp.zeros((ROWS_PT, FEATS), jnp.float32)
    ones_rows = jnp.ones((CHUNK, FEATS), jnp.float32)

    g = _matmul(feat, W)
    degp = _deg_kernel(dst_p, ones_rows, zeros_rows)

    s0 = _scale0(degp, degp, g)
    r1 = _prop_kernel(s0, src_p, dst_p, zeros_rows)
    s1 = _scale_mid(degp, degp, r1, r1)
    r2 = _prop_kernel(s1, src_p, dst_p, zeros_rows)
    out = _final(degp, degp, r2, r2, b.reshape(1, FEATS))
    return out
